# Initial kernel scaffold; baseline (speedup 1.0000x reference)
import jax
import jax.numpy as jnp
from jax.experimental import pallas as pl


def kernel(residue_idx, embed_weight):
    raise NotImplementedError("write your pallas kernel here")


# speedup vs baseline: 5.3707x; 5.3707x over previous
import functools

import jax
import jax.numpy as jnp
from jax import lax
from jax.experimental import pallas as pl
from jax.experimental.pallas import tpu as pltpu
from jax.experimental.pallas import tpu_sc as plsc

_NC = 2
_NS = 16
_NW = _NC * _NS
_D = 128
_CHUNK = 128


@functools.lru_cache(maxsize=None)
def _make_kernel(total):
    b_per_w = total // _NW
    n_chunk = b_per_w // _CHUNK
    mesh = plsc.VectorSubcoreMesh(core_axis_name="c", subcore_axis_name="s")

    @functools.partial(
        pl.kernel,
        out_type=jax.ShapeDtypeStruct((total, _D), jnp.float32),
        mesh=mesh,
        scratch_types=[
            pltpu.VMEM((n_chunk, _CHUNK), jnp.int32),
            pltpu.VMEM((_CHUNK, _D), jnp.float32),
            pltpu.SemaphoreType.DMA,
        ],
    )
    def emb(idx_hbm, table_hbm, out_hbm, idx_v, rows_v, sem):
        wid = lax.axis_index("s") * _NC + lax.axis_index("c")
        base = wid * b_per_w
        pltpu.sync_copy(idx_hbm.at[wid], idx_v)

        def body(j, _):
            pltpu.async_copy(table_hbm.at[idx_v.at[j]], rows_v, sem).wait()
            pltpu.sync_copy(rows_v, out_hbm.at[pl.ds(base + j * _CHUNK, _CHUNK)])
            return 0

        lax.fori_loop(0, n_chunk, body, 0)

    return emb


def kernel(residue_idx, embed_weight):
    bsz, seq = residue_idx.shape
    total = bsz * seq
    n_chunk = total // (_NW * _CHUNK)
    idx3d = residue_idx.astype(jnp.int32).reshape(_NW, n_chunk, _CHUNK)
    out = _make_kernel(total)(idx3d, embed_weight)
    return out.reshape(bsz, seq, _D)


# speedup vs baseline: 6.8176x; 1.2694x over previous
import functools

import jax
import jax.numpy as jnp
from jax import lax
from jax.experimental import pallas as pl
from jax.experimental.pallas import tpu as pltpu
from jax.experimental.pallas import tpu_sc as plsc

_NC = 2
_NS = 16
_NW = _NC * _NS
_D = 128
_CHUNK = 128


@functools.lru_cache(maxsize=None)
def _make_kernel(total):
    b_per_w = total // _NW
    n_chunk = b_per_w // _CHUNK
    mesh = plsc.VectorSubcoreMesh(core_axis_name="c", subcore_axis_name="s")

    nbuf = 5
    assert n_chunk % nbuf == 0
    look = nbuf - 1

    @functools.partial(
        pl.kernel,
        out_type=jax.ShapeDtypeStruct((total, _D), jnp.float32),
        mesh=mesh,
        scratch_types=[
            pltpu.VMEM((n_chunk, _CHUNK), jnp.int32),
            pltpu.VMEM((nbuf, _CHUNK, _D), jnp.float32),
            pltpu.SemaphoreType.DMA((nbuf,)),
            pltpu.SemaphoreType.DMA((nbuf,)),
        ],
    )
    def emb(idx_hbm, table_hbm, out_hbm, idx_v, rows_v, gsem, osem):
        wid = lax.axis_index("s") * _NC + lax.axis_index("c")
        base = wid * b_per_w
        pltpu.sync_copy(idx_hbm.at[wid], idx_v)

        def gather(c, slot):
            return pltpu.make_async_copy(
                table_hbm.at[idx_v.at[c]], rows_v.at[slot], gsem.at[slot])

        def out_copy(c, slot):
            return pltpu.make_async_copy(
                rows_v.at[slot],
                out_hbm.at[pl.ds(base + c * _CHUNK, _CHUNK)],
                osem.at[slot])

        for b in range(look):
            gather(b, b).start()

        def body(g, _):
            for b in range(nbuf):
                j = g * nbuf + b
                gather(j, b).wait()
                out_copy(j, b).start()
                pslot = (b - 1) % nbuf

                @pl.when(j > 0)
                def _():
                    out_copy(j - 1, pslot).wait()

                @pl.when(j + look < n_chunk)
                def _():
                    gather(j + look, pslot).start()
            return 0

        lax.fori_loop(0, n_chunk // nbuf, body, 0)
        out_copy(n_chunk - 1, (n_chunk - 1) % nbuf).wait()

    return emb


def kernel(residue_idx, embed_weight):
    bsz, seq = residue_idx.shape
    total = bsz * seq
    n_chunk = total // (_NW * _CHUNK)
    idx3d = residue_idx.astype(jnp.int32).reshape(_NW, n_chunk, _CHUNK)
    out = _make_kernel(total)(idx3d, embed_weight)
    return out.reshape(bsz, seq, _D)


# speedup vs baseline: 11.7382x; 1.7217x over previous
import functools

import jax
import jax.numpy as jnp
from jax import lax
from jax.experimental import pallas as pl
from jax.experimental.pallas import tpu as pltpu
from jax.experimental.pallas import tpu_sc as plsc

_NC = 2
_NS = 16
_NW = _NC * _NS
_D = 128
_CHUNK = 128


@functools.lru_cache(maxsize=None)
def _make_kernel(total, vocab):
    b_per_w = total // _NW
    n_chunk = b_per_w // _CHUNK
    mesh = plsc.VectorSubcoreMesh(core_axis_name="c", subcore_axis_name="s")

    nbuf = 5
    assert n_chunk % nbuf == 0
    look = nbuf - 1

    @functools.partial(
        pl.kernel,
        out_type=jax.ShapeDtypeStruct((total, _D), jnp.float32),
        mesh=mesh,
        scratch_types=[
            pltpu.VMEM((n_chunk, _CHUNK), jnp.int32),
            pltpu.VMEM((nbuf, _CHUNK, _D), jnp.float32),
            pltpu.VMEM_SHARED((vocab, _D), jnp.float32),
            pltpu.SemaphoreType.DMA((nbuf,)),
            pltpu.SemaphoreType.DMA((nbuf,)),
        ],
    )
    def emb(idx_hbm, table_hbm, out_hbm, idx_v, rows_v, table_sh, gsem, osem):
        sid = lax.axis_index("s")
        wid = sid * _NC + lax.axis_index("c")
        base = wid * b_per_w
        rows_per_tile = (-(-vocab // _NS) + 7) // 8 * 8
        n_full = vocab // rows_per_tile
        tail = vocab - n_full * rows_per_tile

        @pl.when(sid < n_full)
        def _():
            pltpu.sync_copy(
                table_hbm.at[pl.ds(sid * rows_per_tile, rows_per_tile)],
                table_sh.at[pl.ds(sid * rows_per_tile, rows_per_tile)])

        if tail:
            @pl.when(sid == n_full)
            def _():
                pltpu.sync_copy(
                    table_hbm.at[pl.ds(n_full * rows_per_tile, tail)],
                    table_sh.at[pl.ds(n_full * rows_per_tile, tail)])
        pltpu.sync_copy(idx_hbm.at[wid], idx_v)
        plsc.subcore_barrier()

        def gather(c, slot):
            return pltpu.make_async_copy(
                table_sh.at[idx_v.at[c]], rows_v.at[slot], gsem.at[slot])

        def out_copy(c, slot):
            return pltpu.make_async_copy(
                rows_v.at[slot],
                out_hbm.at[pl.ds(base + c * _CHUNK, _CHUNK)],
                osem.at[slot])

        for b in range(look):
            gather(b, b).start()

        def body(g, _):
            for b in range(nbuf):
                j = g * nbuf + b
                gather(j, b).wait()
                out_copy(j, b).start()
                pslot = (b - 1) % nbuf

                @pl.when(j > 0)
                def _():
                    out_copy(j - 1, pslot).wait()

                @pl.when(j + look < n_chunk)
                def _():
                    gather(j + look, pslot).start()
            return 0

        lax.fori_loop(0, n_chunk // nbuf, body, 0)
        out_copy(n_chunk - 1, (n_chunk - 1) % nbuf).wait()

    return emb


def kernel(residue_idx, embed_weight):
    bsz, seq = residue_idx.shape
    total = bsz * seq
    n_chunk = total // (_NW * _CHUNK)
    idx3d = residue_idx.astype(jnp.int32).reshape(_NW, n_chunk, _CHUNK)
    out = _make_kernel(total, embed_weight.shape[0])(idx3d, embed_weight)
    return out.reshape(bsz, seq, _D)
